# Initial kernel scaffold; baseline (speedup 1.0000x reference)
import jax
import jax.numpy as jnp
from jax.experimental import pallas as pl
from jax.experimental.pallas import tpu as pltpu
from jax.experimental.pallas import tpu_sc as plsc

BATCH = 16384
HIST = 50
EMBEDDING_DIM = 32
NUM_INDICES = BATCH * HIST

WINDOW = 128


def _gather_fn(num_indices, value_dim, dtype):
  mesh = plsc.VectorSubcoreMesh(core_axis_name="core",
                                subcore_axis_name="subcore")

  @jax.jit
  def run(weight, idx_flat):
    idx2d = idx_flat.reshape(1, num_indices)

    @pl.kernel(
        out_type=jax.ShapeDtypeStruct((num_indices, value_dim), dtype),
        mesh=mesh,
    )
    def kern(w_hbm, i_hbm, o_hbm):
      def body(i_vmem, o_vmem):
        pltpu.sync_copy(w_hbm.at[i_vmem.at[0]], o_vmem)

      pltpu.emit_pipeline(
          body,
          grid=(num_indices // WINDOW,),
          in_specs=[pl.BlockSpec((1, WINDOW), index_map=lambda i: (0, i))],
          out_specs=[pl.BlockSpec((WINDOW, value_dim),
                                  index_map=lambda i: (i, 0))],
          core_axis_name=("core", "subcore"),
          dimension_semantics=(pltpu.PARALLEL,),
      )(i_hbm, o_hbm)

    return kern(weight, idx2d)

  return run


_RUN = _gather_fn(NUM_INDICES, EMBEDDING_DIM, jnp.float32)


def kernel(x, weight):
  out = _RUN(weight, x.reshape(-1).astype(jnp.int32))
  return out.reshape(x.shape[0], x.shape[1], EMBEDDING_DIM)


# speedup vs baseline: 1.0423x; 1.0423x over previous
import jax
import jax.numpy as jnp
from jax.experimental import pallas as pl
from jax.experimental.pallas import tpu as pltpu
from jax.experimental.pallas import tpu_sc as plsc

BATCH = 16384
HIST = 50
EMBEDDING_DIM = 32
NUM_INDICES = BATCH * HIST

WINDOW = 128


def _gather_fn(num_indices, value_dim, dtype):
  mesh = plsc.VectorSubcoreMesh(core_axis_name="core",
                                subcore_axis_name="subcore")

  @jax.jit
  def run(weight, idx_flat):
    idx2d = idx_flat.reshape(1, num_indices)

    @pl.kernel(
        out_type=jax.ShapeDtypeStruct((num_indices, value_dim), dtype),
        mesh=mesh,
        compiler_params=pltpu.CompilerParams(use_tc_tiling_on_sc=False),
    )
    def kern(w_hbm, i_hbm, o_hbm):
      def body(i_vmem, o_vmem):
        pltpu.sync_copy(w_hbm.at[i_vmem.at[0]], o_vmem)

      pltpu.emit_pipeline(
          body,
          grid=(num_indices // WINDOW,),
          in_specs=[pl.BlockSpec((1, WINDOW), index_map=lambda i: (0, i))],
          out_specs=[pl.BlockSpec((WINDOW, value_dim),
                                  index_map=lambda i: (i, 0))],
          core_axis_name=("core", "subcore"),
          dimension_semantics=(pltpu.PARALLEL,),
      )(i_hbm, o_hbm)

    return kern(weight, idx2d)

  return run


_RUN = _gather_fn(NUM_INDICES, EMBEDDING_DIM, jnp.float32)


def kernel(x, weight):
  out = _RUN(weight, x.reshape(-1).astype(jnp.int32))
  return out.reshape(x.shape[0], x.shape[1], EMBEDDING_DIM)


# speedup vs baseline: 1.0997x; 1.0550x over previous
import jax
import jax.numpy as jnp
from jax.experimental import pallas as pl
from jax.experimental.pallas import tpu as pltpu
from jax.experimental.pallas import tpu_sc as plsc

BATCH = 16384
HIST = 50
EMBEDDING_DIM = 32
NUM_INDICES = BATCH * HIST

WINDOW = 512


def _gather_fn(num_indices, value_dim, dtype):
  mesh = plsc.VectorSubcoreMesh(core_axis_name="core",
                                subcore_axis_name="subcore")

  @jax.jit
  def run(weight, idx_flat):
    idx2d = idx_flat.reshape(1, num_indices)

    @pl.kernel(
        out_type=jax.ShapeDtypeStruct((num_indices, value_dim), dtype),
        mesh=mesh,
        compiler_params=pltpu.CompilerParams(use_tc_tiling_on_sc=False),
    )
    def kern(w_hbm, i_hbm, o_hbm):
      def body(i_vmem, o_vmem):
        pltpu.sync_copy(w_hbm.at[i_vmem.at[0]], o_vmem)

      pltpu.emit_pipeline(
          body,
          grid=(num_indices // WINDOW,),
          in_specs=[pl.BlockSpec((1, WINDOW), index_map=lambda i: (0, i))],
          out_specs=[pl.BlockSpec((WINDOW, value_dim),
                                  index_map=lambda i: (i, 0))],
          core_axis_name=("core", "subcore"),
          dimension_semantics=(pltpu.PARALLEL,),
      )(i_hbm, o_hbm)

    return kern(weight, idx2d)

  return run


_RUN = _gather_fn(NUM_INDICES, EMBEDDING_DIM, jnp.float32)


def kernel(x, weight):
  out = _RUN(weight, x.reshape(-1).astype(jnp.int32))
  return out.reshape(x.shape[0], x.shape[1], EMBEDDING_DIM)


# speedup vs baseline: 1.3571x; 1.2341x over previous
import jax
import jax.numpy as jnp
from jax.experimental import pallas as pl
from jax.experimental.pallas import tpu as pltpu
from jax.experimental.pallas import tpu_sc as plsc

BATCH = 16384
HIST = 50
EMBEDDING_DIM = 32
NUM_INDICES = BATCH * HIST

WINDOW = 400
NUM_WINDOWS = NUM_INDICES // WINDOW


def _gather_fn(num_indices, value_dim, dtype):
  mesh = plsc.VectorSubcoreMesh(core_axis_name="core",
                                subcore_axis_name="subcore")

  @jax.jit
  def run(weight, idx_flat):
    idx3d = idx_flat.reshape(NUM_WINDOWS, 1, WINDOW)

    @pl.kernel(
        out_type=jax.ShapeDtypeStruct((NUM_WINDOWS, WINDOW, value_dim),
                                      dtype),
        mesh=mesh,
        compiler_params=pltpu.CompilerParams(use_tc_tiling_on_sc=False),
    )
    def kern(w_hbm, i_hbm, o_hbm):
      def body(i_vmem, o_vmem):
        pltpu.sync_copy(w_hbm.at[i_vmem.at[0, 0]], o_vmem.at[0])

      pltpu.emit_pipeline(
          body,
          grid=(NUM_WINDOWS,),
          in_specs=[pl.BlockSpec((1, 1, WINDOW),
                                 index_map=lambda i: (i, 0, 0))],
          out_specs=[pl.BlockSpec((1, WINDOW, value_dim),
                                  index_map=lambda i: (i, 0, 0))],
          core_axis_name=("core", "subcore"),
          dimension_semantics=(pltpu.PARALLEL,),
      )(i_hbm, o_hbm)

    return kern(weight, idx3d)

  return run


_RUN = _gather_fn(NUM_INDICES, EMBEDDING_DIM, jnp.float32)


def kernel(x, weight):
  out = _RUN(weight, x.reshape(-1).astype(jnp.int32))
  return out.reshape(x.shape[0], x.shape[1], EMBEDDING_DIM)


# speedup vs baseline: 1.4452x; 1.0649x over previous
import jax
import jax.numpy as jnp
from jax import lax
from jax.experimental import pallas as pl
from jax.experimental.pallas import tpu as pltpu
from jax.experimental.pallas import tpu_sc as plsc

BATCH = 16384
HIST = 50
EMBEDDING_DIM = 32
NUM_INDICES = BATCH * HIST

WINDOW = 128
IT = BATCH // WINDOW
CT = EMBEDDING_DIM // 8
NUM_WINDOWS = HIST * IT


def _gather_fn(dtype):
  mesh = plsc.VectorSubcoreMesh(core_axis_name="core",
                                subcore_axis_name="subcore")

  @jax.jit
  def run(weight, xt):
    idx3d = xt.reshape(NUM_WINDOWS, 1, WINDOW)

    @pl.kernel(
        out_type=jax.ShapeDtypeStruct((HIST, CT, IT, 8, WINDOW), dtype),
        mesh=mesh,
        scratch_types=[pltpu.VMEM((WINDOW, EMBEDDING_DIM), dtype)],
        compiler_params=pltpu.CompilerParams(use_tc_tiling_on_sc=False,
                                             needs_layout_passes=False),
    )
    def kern(w_hbm, i_hbm, o_hbm, rows_v):
      def body(i_vmem, o_vmem):
        pltpu.sync_copy(w_hbm.at[i_vmem.at[0, 0]], rows_v)
        lanes = lax.iota(jnp.int32, 16)
        for lb in range(WINDOW // 16):
          row_idx = lanes + (lb * 16)
          for ct in range(CT):
            for cs in range(8):
              col_idx = jnp.full((16,), ct * 8 + cs, jnp.int32)
              v = plsc.load_gather(rows_v, [row_idx, col_idx])
              o_vmem[0, ct, 0, cs, pl.ds(lb * 16, 16)] = v

      pltpu.emit_pipeline(
          body,
          grid=(NUM_WINDOWS,),
          in_specs=[pl.BlockSpec((1, 1, WINDOW),
                                 index_map=lambda w: (w, 0, 0))],
          out_specs=[pl.BlockSpec((1, CT, 1, 8, WINDOW),
                                  index_map=lambda w: (w // IT, 0, w % IT,
                                                       0, 0))],
          core_axis_name=("core", "subcore"),
          dimension_semantics=(pltpu.PARALLEL,),
      )(i_hbm, o_hbm)

    return kern(weight, idx3d)

  return run


_RUN = _gather_fn(jnp.float32)


def kernel(x, weight):
  out5 = _RUN(weight, x.astype(jnp.int32).T)
  return out5.transpose(2, 4, 0, 1, 3).reshape(BATCH, HIST, EMBEDDING_DIM)


# speedup vs baseline: 1.4458x; 1.0004x over previous
import jax
import jax.numpy as jnp
from jax import lax
from jax.experimental import pallas as pl
from jax.experimental.pallas import tpu as pltpu
from jax.experimental.pallas import tpu_sc as plsc

BATCH = 16384
HIST = 50
EMBEDDING_DIM = 32
NUM_INDICES = BATCH * HIST

WINDOW = 128
IT = BATCH // WINDOW
CT = EMBEDDING_DIM // 8
NUM_WINDOWS = HIST * IT


def _gather_fn(dtype):
  mesh = plsc.VectorSubcoreMesh(core_axis_name="core",
                                subcore_axis_name="subcore")

  @jax.jit
  def run(weight, xt):
    idx3d = xt.reshape(NUM_WINDOWS, 1, WINDOW)

    @pl.kernel(
        out_type=jax.ShapeDtypeStruct((HIST, CT, IT, 8, WINDOW), dtype),
        mesh=mesh,
        scratch_types=[pltpu.VMEM((WINDOW, EMBEDDING_DIM), dtype)],
        compiler_params=pltpu.CompilerParams(use_tc_tiling_on_sc=False,
                                             needs_layout_passes=False,
                                             disable_bounds_checks=True),
    )
    def kern(w_hbm, i_hbm, o_hbm, rows_v):
      lanes = lax.iota(jnp.int32, 16)
      row_vecs = [lanes + (lb * 16) for lb in range(WINDOW // 16)]
      col_vecs = [jnp.full((16,), c, jnp.int32) for c in range(EMBEDDING_DIM)]

      def body(i_vmem, o_vmem):
        pltpu.sync_copy(w_hbm.at[i_vmem.at[0, 0]], rows_v)
        for lb in range(WINDOW // 16):
          for ct in range(CT):
            for cs in range(8):
              v = plsc.load_gather(rows_v, [row_vecs[lb], col_vecs[ct * 8 + cs]])
              o_vmem[0, ct, 0, cs, pl.ds(lb * 16, 16)] = v

      pltpu.emit_pipeline(
          body,
          grid=(NUM_WINDOWS,),
          in_specs=[pl.BlockSpec((1, 1, WINDOW),
                                 index_map=lambda w: (w, 0, 0))],
          out_specs=[pl.BlockSpec((1, CT, 1, 8, WINDOW),
                                  index_map=lambda w: (w // IT, 0, w % IT,
                                                       0, 0))],
          core_axis_name=("core", "subcore"),
          dimension_semantics=(pltpu.PARALLEL,),
      )(i_hbm, o_hbm)

    return kern(weight, idx3d)

  return run


_RUN = _gather_fn(jnp.float32)


def kernel(x, weight):
  out5 = _RUN(weight, x.astype(jnp.int32).T)
  return out5.transpose(2, 4, 0, 1, 3).reshape(BATCH, HIST, EMBEDDING_DIM)


# speedup vs baseline: 1.5348x; 1.0615x over previous
import jax
import jax.numpy as jnp
from jax import lax
from jax.experimental import pallas as pl
from jax.experimental.pallas import tpu as pltpu
from jax.experimental.pallas import tpu_sc as plsc

BATCH = 16384
HIST = 50
EMBEDDING_DIM = 32
NUM_INDICES = BATCH * HIST

WINDOW = 512
ITPW = WINDOW // 128
IT = BATCH // 128
CT = EMBEDDING_DIM // 8
NUM_WINDOWS = NUM_INDICES // WINDOW


def _gather_fn(dtype):
  mesh = plsc.VectorSubcoreMesh(core_axis_name="core",
                                subcore_axis_name="subcore")

  @jax.jit
  def run(weight, xt):
    idx3d = xt.reshape(NUM_WINDOWS, 1, WINDOW)
    wpj = IT // ITPW

    @pl.kernel(
        out_type=jax.ShapeDtypeStruct((HIST, CT, IT, 8, 128), dtype),
        mesh=mesh,
        scratch_types=[pltpu.VMEM((WINDOW, EMBEDDING_DIM), dtype)],
        compiler_params=pltpu.CompilerParams(use_tc_tiling_on_sc=False,
                                             needs_layout_passes=False,
                                             disable_bounds_checks=True),
    )
    def kern(w_hbm, i_hbm, o_hbm, rows_v):
      lanes = lax.iota(jnp.int32, 16)
      row_vecs = [lanes + (r16 * 16) for r16 in range(WINDOW // 16)]
      col_vecs = [jnp.full((16,), c, jnp.int32) for c in range(EMBEDDING_DIM)]

      def body(i_vmem, o_vmem):
        pltpu.sync_copy(w_hbm.at[i_vmem.at[0, 0]], rows_v)
        for itx in range(ITPW):
          for lb in range(8):
            for ct in range(CT):
              for cs in range(8):
                v = plsc.load_gather(
                    rows_v,
                    [row_vecs[itx * 8 + lb], col_vecs[ct * 8 + cs]])
                o_vmem[0, ct, itx, cs, pl.ds(lb * 16, 16)] = v

      pltpu.emit_pipeline(
          body,
          grid=(NUM_WINDOWS,),
          in_specs=[pl.BlockSpec((1, 1, WINDOW),
                                 index_map=lambda w: (w, 0, 0))],
          out_specs=[pl.BlockSpec((1, CT, ITPW, 8, 128),
                                  index_map=lambda w: (w // wpj, 0,
                                                       w % wpj, 0, 0))],
          core_axis_name=("core", "subcore"),
          dimension_semantics=(pltpu.PARALLEL,),
      )(i_hbm, o_hbm)

    return kern(weight, idx3d)

  return run


_RUN = _gather_fn(jnp.float32)


def kernel(x, weight):
  out5 = _RUN(weight, x.astype(jnp.int32).T)
  return out5.transpose(2, 4, 0, 1, 3).reshape(BATCH, HIST, EMBEDDING_DIM)


# speedup vs baseline: 1.5439x; 1.0060x over previous
import jax
import jax.numpy as jnp
from jax import lax
from jax.experimental import pallas as pl
from jax.experimental.pallas import tpu as pltpu
from jax.experimental.pallas import tpu_sc as plsc

BATCH = 16384
HIST = 50
EMBEDDING_DIM = 32
NUM_INDICES = BATCH * HIST

WINDOW = 1024
ITPW = WINDOW // 128
IT = BATCH // 128
CT = EMBEDDING_DIM // 8
NUM_WINDOWS = NUM_INDICES // WINDOW


def _gather_fn(dtype):
  mesh = plsc.VectorSubcoreMesh(core_axis_name="core",
                                subcore_axis_name="subcore")

  @jax.jit
  def run(weight, xt):
    idx3d = xt.reshape(NUM_WINDOWS, 1, WINDOW)
    wpj = IT // ITPW

    @pl.kernel(
        out_type=jax.ShapeDtypeStruct((HIST, CT, IT, 8, 128), dtype),
        mesh=mesh,
        scratch_types=[pltpu.VMEM((WINDOW, EMBEDDING_DIM), dtype)],
        compiler_params=pltpu.CompilerParams(use_tc_tiling_on_sc=False,
                                             needs_layout_passes=False,
                                             disable_bounds_checks=True),
    )
    def kern(w_hbm, i_hbm, o_hbm, rows_v):
      lanes = lax.iota(jnp.int32, 16)
      row_vecs = [lanes + (r16 * 16) for r16 in range(WINDOW // 16)]
      col_vecs = [jnp.full((16,), c, jnp.int32) for c in range(EMBEDDING_DIM)]

      def body(i_vmem, o_vmem):
        pltpu.sync_copy(w_hbm.at[i_vmem.at[0, 0]], rows_v)
        for itx in range(ITPW):
          for lb in range(8):
            for ct in range(CT):
              for cs in range(8):
                v = plsc.load_gather(
                    rows_v,
                    [row_vecs[itx * 8 + lb], col_vecs[ct * 8 + cs]])
                o_vmem[0, ct, itx, cs, pl.ds(lb * 16, 16)] = v

      pltpu.emit_pipeline(
          body,
          grid=(NUM_WINDOWS,),
          in_specs=[pl.BlockSpec((1, 1, WINDOW),
                                 index_map=lambda w: (w, 0, 0))],
          out_specs=[pl.BlockSpec((1, CT, ITPW, 8, 128),
                                  index_map=lambda w: (w // wpj, 0,
                                                       w % wpj, 0, 0))],
          core_axis_name=("core", "subcore"),
          dimension_semantics=(pltpu.PARALLEL,),
      )(i_hbm, o_hbm)

    return kern(weight, idx3d)

  return run


_RUN = _gather_fn(jnp.float32)


def kernel(x, weight):
  out5 = _RUN(weight, x.astype(jnp.int32).T)
  return out5.transpose(2, 4, 0, 1, 3).reshape(BATCH, HIST, EMBEDDING_DIM)
